# baseline (device time: 10368 ns/iter reference)
import jax
import jax.numpy as jnp
from jax import lax
from jax.experimental import pallas as pl
from jax.experimental.pallas import tpu as pltpu

N_DEV = 16
N_PLANE = 4


def _fold4(recv_ref):
    vals = recv_ref[:, 0, :]
    idxs = recv_ref[:, 1, :]
    m = jnp.max(vals, axis=0)
    i = jnp.min(jnp.where(vals == m[None, :], idxs, jnp.float32(1e9)), axis=0)
    return m, i


def kernel(x):
    m_per, n = x.shape

    def body(x_ref, out_ref, send1, recv1, send2, recv2,
             ssem1, rsem1, ssem2, rsem2, col_sem):
        my = lax.axis_index("i")
        s = my % N_PLANE
        base = my - s
        z = my // N_PLANE

        def plane_peer(o):
            return base + (s + o) % N_PLANE

        def col_peer(o):
            return s + N_PLANE * ((z + o) % N_PLANE)

        barrier_sem = pltpu.get_barrier_semaphore()
        for o in range(1, N_PLANE):
            pl.semaphore_signal(
                barrier_sem, inc=1,
                device_id=(plane_peer(o),),
                device_id_type=pl.DeviceIdType.MESH,
            )
            pl.semaphore_signal(
                col_sem, inc=1,
                device_id=(col_peer(o),),
                device_id_type=pl.DeviceIdType.MESH,
            )

        xv = x_ref[:, :]
        val = jnp.max(xv, axis=0)
        row_ids = lax.broadcasted_iota(jnp.int32, (m_per, n), 0)
        loc_idx = jnp.min(
            jnp.where(xv == val[None, :], row_ids, m_per), axis=0
        )
        gidx = (loc_idx + my * m_per).astype(jnp.float32)
        send1[0, :] = val
        send1[1, :] = gidx
        recv1[0, 0, :] = val
        recv1[0, 1, :] = gidx

        pl.semaphore_wait(barrier_sem, N_PLANE - 1)
        rdmas1 = []
        for o in range(1, N_PLANE):
            r = pltpu.make_async_remote_copy(
                src_ref=send1,
                dst_ref=recv1.at[o],
                send_sem=ssem1.at[o],
                recv_sem=rsem1.at[o],
                device_id=(plane_peer(o),),
                device_id_type=pl.DeviceIdType.MESH,
            )
            r.start()
            rdmas1.append(r)
        for r in rdmas1:
            r.wait_recv()
        pval, pidx = _fold4(recv1)
        send2[0, :] = pval
        send2[1, :] = pidx
        recv2[0, 0, :] = pval
        recv2[0, 1, :] = pidx

        pl.semaphore_wait(col_sem, N_PLANE - 1)
        rdmas2 = []
        for o in range(1, N_PLANE):
            r = pltpu.make_async_remote_copy(
                src_ref=send2,
                dst_ref=recv2.at[o],
                send_sem=ssem2.at[o],
                recv_sem=rsem2.at[o],
                device_id=(col_peer(o),),
                device_id_type=pl.DeviceIdType.MESH,
            )
            r.start()
            rdmas2.append(r)
        for r in rdmas2:
            r.wait_recv()
        gmax, gi = _fold4(recv2)
        out_ref[0, :] = gmax
        out_ref[1, :] = gi

        for r in rdmas1:
            r.wait_send()
        for r in rdmas2:
            r.wait_send()

    return pl.pallas_call(
        body,
        out_shape=jax.ShapeDtypeStruct((2, n), jnp.float32),
        in_specs=[pl.BlockSpec(memory_space=pltpu.VMEM)],
        out_specs=pl.BlockSpec(memory_space=pltpu.VMEM),
        scratch_shapes=[
            pltpu.VMEM((2, n), jnp.float32),
            pltpu.VMEM((N_PLANE, 2, n), jnp.float32),
            pltpu.VMEM((2, n), jnp.float32),
            pltpu.VMEM((N_PLANE, 2, n), jnp.float32),
            pltpu.SemaphoreType.DMA((N_PLANE,)),
            pltpu.SemaphoreType.DMA((N_PLANE,)),
            pltpu.SemaphoreType.DMA((N_PLANE,)),
            pltpu.SemaphoreType.DMA((N_PLANE,)),
            pltpu.SemaphoreType.REGULAR,
        ],
        compiler_params=pltpu.CompilerParams(collective_id=0),
    )(x)


# device time: 9316 ns/iter; 1.1129x vs baseline; 1.1129x over previous
import jax
import jax.numpy as jnp
from jax import lax
from jax.experimental import pallas as pl
from jax.experimental.pallas import tpu as pltpu

N_DEV = 16


def kernel(x):
    m_per, n = x.shape

    def body(x_hbm, out_ref, x_vmem, send_ref, recv_ref,
             copy_sem, send_sems, recv_sems):
        my = lax.axis_index("i")

        barrier_sem = pltpu.get_barrier_semaphore()
        for e in range(1, N_DEV):
            pl.semaphore_signal(
                barrier_sem, inc=1,
                device_id=((my + e) % N_DEV,),
                device_id_type=pl.DeviceIdType.MESH,
            )

        cp = pltpu.make_async_copy(x_hbm, x_vmem, copy_sem)
        cp.start()
        cp.wait()

        xv = x_vmem[:, :]
        val = jnp.max(xv, axis=0)
        row_ids = lax.broadcasted_iota(jnp.int32, (m_per, n), 0)
        loc_idx = jnp.min(
            jnp.where(xv == val[None, :], row_ids, m_per), axis=0
        )
        gidx = (loc_idx + my * m_per).astype(jnp.float32)
        send_ref[0, :] = val
        send_ref[1, :] = gidx
        recv_ref[0, 0, :] = val
        recv_ref[0, 1, :] = gidx

        pl.semaphore_wait(barrier_sem, N_DEV - 1)

        rdmas = []
        for e in range(1, N_DEV):
            rdma = pltpu.make_async_remote_copy(
                src_ref=send_ref,
                dst_ref=recv_ref.at[e],
                send_sem=send_sems.at[e],
                recv_sem=recv_sems.at[e],
                device_id=((my + e) % N_DEV,),
                device_id_type=pl.DeviceIdType.MESH,
            )
            rdma.start()
            rdmas.append(rdma)

        for r in rdmas:
            r.wait_recv()

        vals = recv_ref[:, 0, :]
        idxs = recv_ref[:, 1, :]
        gmax = jnp.max(vals, axis=0)
        gidx = jnp.min(
            jnp.where(vals == gmax[None, :], idxs, jnp.float32(1e9)), axis=0
        )
        out_ref[0, :] = gmax
        out_ref[1, :] = gidx

        for r in rdmas:
            r.wait_send()

    return pl.pallas_call(
        body,
        out_shape=jax.ShapeDtypeStruct((2, n), jnp.float32),
        in_specs=[pl.BlockSpec(memory_space=pl.ANY)],
        out_specs=pl.BlockSpec(memory_space=pltpu.VMEM),
        scratch_shapes=[
            pltpu.VMEM((m_per, n), jnp.float32),
            pltpu.VMEM((2, n), jnp.float32),
            pltpu.VMEM((N_DEV, 2, n), jnp.float32),
            pltpu.SemaphoreType.DMA,
            pltpu.SemaphoreType.DMA((N_DEV,)),
            pltpu.SemaphoreType.DMA((N_DEV,)),
        ],
        compiler_params=pltpu.CompilerParams(collective_id=0),
    )(x)
